# baseline (speedup 1.0000x reference)
import functools

import jax
import jax.numpy as jnp
from jax import lax
from jax.experimental import pallas as pl
from jax.experimental.pallas import tpu as pltpu
from jax.experimental.pallas import tpu_sc as plsc

L = 16
NC, NS = 2, 16
NW = NC * NS
TABLE_SIZE = 128
ROWS_C = 64


def _make_sc_lookup(batch: int, hist: int):
    rows_w = batch // NW
    n_chunks = rows_w // ROWS_C
    n_win = (hist + L - 1) // L
    last_off = hist - L
    mesh = plsc.VectorSubcoreMesh(core_axis_name="c", subcore_axis_name="s")

    @functools.partial(
        pl.kernel,
        mesh=mesh,
        out_type=jax.ShapeDtypeStruct((batch, hist), jnp.int32),
        scratch_types=[
            [pltpu.VMEM((ROWS_C, hist), jnp.int32) for _ in range(2)],
            [pltpu.VMEM((ROWS_C, hist), jnp.int32) for _ in range(2)],
            pltpu.VMEM((TABLE_SIZE,), jnp.int32),
            pltpu.VMEM((TABLE_SIZE,), jnp.int32),
            pltpu.VMEM((TABLE_SIZE,), jnp.int32),
            [pltpu.SemaphoreType.DMA for _ in range(2)],
            [pltpu.SemaphoreType.DMA for _ in range(2)],
        ],
        compiler_params=pltpu.CompilerParams(needs_layout_passes=False, skip_device_barrier=True),
    )
    def lookup(ids_hbm, keys_hbm, vals_hbm, out_hbm,
               ibufs, obufs, inv, kbuf, vbuf, in_sems, out_sems):
        wid = lax.axis_index("s") * NC + lax.axis_index("c")
        row0 = wid * rows_w

        def in_copy(c):
            return pltpu.make_async_copy(
                ids_hbm.at[pl.ds(row0 + c * ROWS_C, ROWS_C), :],
                ibufs[c % 2], in_sems[c % 2])

        def out_copy(c):
            return pltpu.make_async_copy(
                obufs[c % 2],
                out_hbm.at[pl.ds(row0 + c * ROWS_C, ROWS_C), :],
                out_sems[c % 2])

        in_copy(0).start()
        in_copy(1).start()
        pltpu.sync_copy(keys_hbm, kbuf)
        pltpu.sync_copy(vals_hbm, vbuf)
        for i in range(TABLE_SIZE // L):
            inv[pl.ds(i * L, L)] = jnp.zeros((L,), jnp.int32)
        for i in range(TABLE_SIZE // L):
            sl = pl.ds(i * L, L)
            plsc.store_scatter(inv, [kbuf[sl]], vbuf[sl])

        for c in range(n_chunks):
            in_copy(c).wait()
            if c >= 2:
                out_copy(c - 2).wait()
            ibuf, obuf = ibufs[c % 2], obufs[c % 2]

            @plsc.parallel_loop(0, ROWS_C, step=1, unroll=2)
            def _(r):
                for w in range(n_win):
                    off = last_off if w == n_win - 1 else w * L
                    sl = pl.ds(off, L)
                    obuf[r, sl] = plsc.load_gather(inv, [ibuf[r, sl]])

            out_copy(c).start()
            if c + 2 < n_chunks:
                in_copy(c + 2).start()

        out_copy(n_chunks - 2).wait()
        out_copy(n_chunks - 1).wait()

    return lookup


def kernel(inputs, keys, vals):
    batch, hist = inputs.shape
    pad = TABLE_SIZE - keys.shape[0]
    keys_p = jnp.concatenate(
        [keys.astype(jnp.int32), jnp.full((pad,), TABLE_SIZE - 1, jnp.int32)])
    vals_p = jnp.concatenate([vals.astype(jnp.int32), jnp.zeros((pad,), jnp.int32)])
    return _make_sc_lookup(batch, hist)(inputs, keys_p, vals_p)


# speedup vs baseline: 1.9375x; 1.9375x over previous
import functools

import jax
import jax.numpy as jnp
from jax import lax
from jax.experimental import pallas as pl
from jax.experimental.pallas import tpu as pltpu
from jax.experimental.pallas import tpu_sc as plsc

L = 16
NC, NS = 2, 16
NW = NC * NS
TABLE_SIZE = 128
N_CHUNKS = 8


def _make_sc_lookup(n_total: int, n_keys: int):
    per_w = n_total // NW
    chunk = per_w // N_CHUNKS
    mesh = plsc.VectorSubcoreMesh(core_axis_name="c", subcore_axis_name="s")

    @functools.partial(
        pl.kernel,
        mesh=mesh,
        out_type=jax.ShapeDtypeStruct((n_total,), jnp.int32),
        scratch_types=[
            [pltpu.VMEM((chunk,), jnp.int32) for _ in range(2)],
            [pltpu.VMEM((chunk,), jnp.int32) for _ in range(2)],
            pltpu.VMEM((TABLE_SIZE,), jnp.int32),
            pltpu.VMEM((TABLE_SIZE,), jnp.int32),
            pltpu.VMEM((TABLE_SIZE,), jnp.int32),
            [pltpu.SemaphoreType.DMA for _ in range(2)],
            [pltpu.SemaphoreType.DMA for _ in range(2)],
        ],
        compiler_params=pltpu.CompilerParams(needs_layout_passes=False),
    )
    def lookup(ids_hbm, keys_hbm, vals_hbm, out_hbm,
               ibufs, obufs, inv, kbuf, vbuf, in_sems, out_sems):
        wid = lax.axis_index("s") * NC + lax.axis_index("c")
        base = wid * per_w

        def in_copy(c):
            return pltpu.make_async_copy(
                ids_hbm.at[pl.ds(base + c * chunk, chunk)],
                ibufs[c % 2], in_sems[c % 2])

        def out_copy(c):
            return pltpu.make_async_copy(
                obufs[c % 2],
                out_hbm.at[pl.ds(base + c * chunk, chunk)],
                out_sems[c % 2])

        in_copy(0).start()
        in_copy(1).start()
        for i in range(TABLE_SIZE // L):
            sl = pl.ds(i * L, L)
            kbuf[sl] = jnp.full((L,), TABLE_SIZE - 1, jnp.int32)
            vbuf[sl] = jnp.zeros((L,), jnp.int32)
            inv[sl] = jnp.zeros((L,), jnp.int32)
        pltpu.sync_copy(keys_hbm, kbuf.at[pl.ds(0, n_keys)])
        pltpu.sync_copy(vals_hbm, vbuf.at[pl.ds(0, n_keys)])
        for i in range(TABLE_SIZE // L):
            sl = pl.ds(i * L, L)
            plsc.store_scatter(inv, [kbuf[sl]], vbuf[sl])

        for c in range(N_CHUNKS):
            in_copy(c).wait()
            if c >= 2:
                out_copy(c - 2).wait()
            ibuf, obuf = ibufs[c % 2], obufs[c % 2]

            @plsc.parallel_loop(0, chunk, step=L, unroll=8)
            def _(off):
                sl = pl.ds(off, L)
                obuf[sl] = plsc.load_gather(inv, [ibuf[sl]])

            out_copy(c).start()
            if c + 2 < N_CHUNKS:
                in_copy(c + 2).start()

        out_copy(N_CHUNKS - 2).wait()
        out_copy(N_CHUNKS - 1).wait()

    return lookup


def kernel(inputs, keys, vals):
    batch, hist = inputs.shape
    n_total = batch * hist
    ht, bt = hist // 8, batch // 128
    flat = (inputs.T.reshape(ht, 8, bt, 128)
            .transpose(0, 2, 1, 3).reshape(n_total))
    out = _make_sc_lookup(n_total, keys.shape[0])(
        flat, keys.astype(jnp.int32), vals.astype(jnp.int32))
    return (out.reshape(ht, bt, 8, 128).transpose(0, 2, 1, 3)
            .reshape(hist, batch).T)


# speedup vs baseline: 1.9641x; 1.0137x over previous
import functools

import jax
import jax.numpy as jnp
from jax import lax
from jax.experimental import pallas as pl
from jax.experimental.pallas import tpu as pltpu
from jax.experimental.pallas import tpu_sc as plsc

L = 16
NC, NS = 2, 16
NW = NC * NS
TABLE_SIZE = 128
N_CHUNKS = 16
IN_DEPTH = 4


def _make_sc_lookup(n_total: int, n_keys: int):
    per_w = n_total // NW
    chunk = per_w // N_CHUNKS
    mesh = plsc.VectorSubcoreMesh(core_axis_name="c", subcore_axis_name="s")

    @functools.partial(
        pl.kernel,
        mesh=mesh,
        out_type=jax.ShapeDtypeStruct((n_total,), jnp.int32),
        scratch_types=[
            [pltpu.VMEM((chunk,), jnp.int32) for _ in range(IN_DEPTH)],
            [pltpu.VMEM((chunk,), jnp.int32) for _ in range(2)],
            pltpu.VMEM((TABLE_SIZE,), jnp.int32),
            pltpu.VMEM((TABLE_SIZE,), jnp.int32),
            pltpu.VMEM((TABLE_SIZE,), jnp.int32),
            [pltpu.SemaphoreType.DMA for _ in range(IN_DEPTH)],
            [pltpu.SemaphoreType.DMA for _ in range(2)],
        ],
        compiler_params=pltpu.CompilerParams(needs_layout_passes=False),
    )
    def lookup(ids_hbm, keys_hbm, vals_hbm, out_hbm,
               ibufs, obufs, inv, kbuf, vbuf, in_sems, out_sems):
        wid = lax.axis_index("s") * NC + lax.axis_index("c")
        base = wid * per_w

        def in_copy(c):
            return pltpu.make_async_copy(
                ids_hbm.at[pl.ds(base + c * chunk, chunk)],
                ibufs[c % IN_DEPTH], in_sems[c % IN_DEPTH])

        def out_copy(c):
            return pltpu.make_async_copy(
                obufs[c % 2],
                out_hbm.at[pl.ds(base + c * chunk, chunk)],
                out_sems[c % 2])

        for c in range(IN_DEPTH):
            in_copy(c).start()
        for i in range(TABLE_SIZE // L):
            sl = pl.ds(i * L, L)
            kbuf[sl] = jnp.full((L,), TABLE_SIZE - 1, jnp.int32)
            vbuf[sl] = jnp.zeros((L,), jnp.int32)
            inv[sl] = jnp.zeros((L,), jnp.int32)
        pltpu.sync_copy(keys_hbm, kbuf.at[pl.ds(0, n_keys)])
        pltpu.sync_copy(vals_hbm, vbuf.at[pl.ds(0, n_keys)])
        for i in range(TABLE_SIZE // L):
            sl = pl.ds(i * L, L)
            plsc.store_scatter(inv, [kbuf[sl]], vbuf[sl])

        for c in range(N_CHUNKS):
            in_copy(c).wait()
            if c >= 2:
                out_copy(c - 2).wait()
            ibuf, obuf = ibufs[c % IN_DEPTH], obufs[c % 2]

            @plsc.parallel_loop(0, chunk, step=L, unroll=8)
            def _(off):
                sl = pl.ds(off, L)
                obuf[sl] = plsc.load_gather(inv, [ibuf[sl]])

            out_copy(c).start()
            if c + IN_DEPTH < N_CHUNKS:
                in_copy(c + IN_DEPTH).start()

        out_copy(N_CHUNKS - 2).wait()
        out_copy(N_CHUNKS - 1).wait()

    return lookup


def kernel(inputs, keys, vals):
    batch, hist = inputs.shape
    n_total = batch * hist
    ht, bt = hist // 8, batch // 128
    flat = (inputs.T.reshape(ht, 8, bt, 128)
            .transpose(0, 2, 1, 3).reshape(n_total))
    out = _make_sc_lookup(n_total, keys.shape[0])(
        flat, keys.astype(jnp.int32), vals.astype(jnp.int32))
    return (out.reshape(ht, bt, 8, 128).transpose(0, 2, 1, 3)
            .reshape(hist, batch).T)


# speedup vs baseline: 2.0494x; 1.0434x over previous
import functools

import jax
import jax.numpy as jnp
from jax import lax
from jax.experimental import pallas as pl
from jax.experimental.pallas import tpu as pltpu
from jax.experimental.pallas import tpu_sc as plsc

L = 16
NC, NS = 2, 16
NW = NC * NS
TABLE_SIZE = 128
N_CHUNKS = 16
IN_DEPTH = 4


def _make_sc_lookup(n_total: int, n_keys: int):
    per_w = n_total // NW
    chunk = per_w // N_CHUNKS
    mesh = plsc.VectorSubcoreMesh(core_axis_name="c", subcore_axis_name="s")

    @functools.partial(
        pl.kernel,
        mesh=mesh,
        out_type=jax.ShapeDtypeStruct((n_total,), jnp.int32),
        scratch_types=[
            [pltpu.VMEM((chunk,), jnp.int32) for _ in range(IN_DEPTH)],
            [pltpu.VMEM((chunk,), jnp.int32) for _ in range(2)],
            pltpu.VMEM((TABLE_SIZE,), jnp.int32),
            pltpu.VMEM((TABLE_SIZE,), jnp.int32),
            pltpu.VMEM((TABLE_SIZE,), jnp.int32),
            [pltpu.SemaphoreType.DMA for _ in range(IN_DEPTH)],
            [pltpu.SemaphoreType.DMA for _ in range(2)],
        ],
        compiler_params=pltpu.CompilerParams(needs_layout_passes=False),
    )
    def lookup(ids_hbm, keys_hbm, vals_hbm, out_hbm,
               ibufs, obufs, inv, kbuf, vbuf, in_sems, out_sems):
        wid = lax.axis_index("s") * NC + lax.axis_index("c")
        base = wid * per_w

        def in_copy_d(c, slot):
            return pltpu.make_async_copy(
                ids_hbm.at[pl.ds(base + c * chunk, chunk)],
                ibufs[slot], in_sems[slot])

        def out_copy_d(c, slot):
            return pltpu.make_async_copy(
                obufs[slot],
                out_hbm.at[pl.ds(base + c * chunk, chunk)],
                out_sems[slot])

        for c in range(IN_DEPTH):
            in_copy_d(c, c).start()
        for i in range(TABLE_SIZE // L):
            sl = pl.ds(i * L, L)
            kbuf[sl] = jnp.full((L,), TABLE_SIZE - 1, jnp.int32)
            vbuf[sl] = jnp.zeros((L,), jnp.int32)
            inv[sl] = jnp.zeros((L,), jnp.int32)
        pltpu.sync_copy(keys_hbm, kbuf.at[pl.ds(0, n_keys)])
        pltpu.sync_copy(vals_hbm, vbuf.at[pl.ds(0, n_keys)])
        for i in range(TABLE_SIZE // L):
            sl = pl.ds(i * L, L)
            plsc.store_scatter(inv, [kbuf[sl]], vbuf[sl])

        @pl.loop(0, N_CHUNKS, step=IN_DEPTH)
        def _(g):
            for j in range(IN_DEPTH):
                c = g + j
                in_copy_d(c, j).wait()
                @pl.when(c >= 2)
                def _():
                    out_copy_d(c - 2, (j - 2) % 2).wait()
                ibuf, obuf = ibufs[j], obufs[j % 2]

                @plsc.parallel_loop(0, chunk, step=L, unroll=8)
                def _(off):
                    sl = pl.ds(off, L)
                    obuf[sl] = plsc.load_gather(inv, [ibuf[sl]])

                out_copy_d(c, j % 2).start()
                @pl.when(c + IN_DEPTH < N_CHUNKS)
                def _():
                    in_copy_d(c + IN_DEPTH, j).start()

        out_copy_d(N_CHUNKS - 2, (N_CHUNKS - 2) % 2).wait()
        out_copy_d(N_CHUNKS - 1, (N_CHUNKS - 1) % 2).wait()

    return lookup


def kernel(inputs, keys, vals):
    batch, hist = inputs.shape
    n_total = batch * hist
    ht, bt = hist // 8, batch // 128
    flat = (inputs.T.reshape(ht, 8, bt, 128)
            .transpose(0, 2, 1, 3).reshape(n_total))
    out = _make_sc_lookup(n_total, keys.shape[0])(
        flat, keys.astype(jnp.int32), vals.astype(jnp.int32))
    return (out.reshape(ht, bt, 8, 128).transpose(0, 2, 1, 3)
            .reshape(hist, batch).T)
